# baseline (speedup 1.0000x reference)
import functools

import jax
import jax.numpy as jnp
from jax import lax
from jax.experimental import pallas as pl
from jax.experimental.pallas import tpu as pltpu
from jax.experimental.pallas import tpu_sc as plsc

N = 10000
D_FEAT = 128
K_CENTERS = 2500
E = 320000
EPS = 1e-3

NC = 2
NS = 16
NW = NC * NS

P1_EDGES = E // NW
P1_CHUNK = 80
P1_NCHUNK = P1_EDGES // P1_CHUNK

MLP1_BLK = 2560
MLP1_ROWS = MLP1_BLK // 4
NBLK = E // MLP1_BLK
SM_SR = 160
SM_SUBS = MLP1_ROWS // SM_SR

SM_FG = 4
SM_EG = NW // SM_FG
FW = D_FEAT // SM_FG
ACC_N = K_CENTERS * FW

K_PAD = 2560


def _mesh():
    return plsc.VectorSubcoreMesh(core_axis_name="c", subcore_axis_name="s")


def _wid():
    return lax.axis_index("s") * NC + lax.axis_index("c")


def _vreg_gather(vec, idx):
    dn = lax.GatherDimensionNumbers(
        offset_dims=(), collapsed_slice_dims=(0,), start_index_map=(0,))
    return lax.gather(vec, idx[:, None], dn, slice_sizes=(1,),
                      mode=lax.GatherScatterMode.PROMISE_IN_BOUNDS)


def _bdot(a, b):
    return jnp.dot(a.astype(jnp.bfloat16), b.astype(jnp.bfloat16),
                   preferred_element_type=jnp.float32)


def _prep_body(pf_ref, pc_ref, w0a_ref, b0_ref, t_ref):
    blk = pf_ref.shape[0]
    g = _bdot(pf_ref[...], w0a_ref[...]) + b0_ref[...]
    t_ref[...] = jnp.concatenate(
        [g, pc_ref[...], jnp.zeros((blk, 61), jnp.float32)], axis=-1)


def _prep(pf, pc, w0a, b0):
    blk = 2000
    grid = N // blk
    return pl.pallas_call(
        _prep_body,
        grid=(grid,),
        in_specs=[
            pl.BlockSpec((blk, D_FEAT), lambda i: (i, 0)),
            pl.BlockSpec((blk, 3), lambda i: (i, 0)),
            pl.BlockSpec((D_FEAT, 64), lambda i: (0, 0)),
            pl.BlockSpec((1, 64), lambda i: (0, 0)),
        ],
        out_specs=pl.BlockSpec((blk, 128), lambda i: (i, 0)),
        out_shape=jax.ShapeDtypeStruct((N, 128), jnp.float32),
    )(pf, pc, w0a, b0)


def _gatherc_body(d_hbm, kidx_hbm, c2_hbm, idx_v, rows_v, sem):
    w = _wid()
    base = w * (K_PAD // NW)
    pltpu.sync_copy(kidx_hbm.at[pl.ds(base, K_PAD // NW)], idx_v)
    pltpu.async_copy(d_hbm.at[idx_v], rows_v, sem).wait()
    pltpu.sync_copy(rows_v, c2_hbm.at[pl.ds(base, K_PAD // NW)])


def _gatherc(d, kidx_pad):
    per = K_PAD // NW
    f = pl.kernel(
        _gatherc_body,
        out_type=jax.ShapeDtypeStruct((K_PAD, 128), jnp.float32),
        mesh=_mesh(),
        scratch_types=[
            pltpu.VMEM((per,), jnp.int32),
            pltpu.VMEM((per, 128), jnp.float32),
            pltpu.SemaphoreType.DMA,
        ],
    )
    return f(d, kidx_pad)


def _round_bf16(x):
    u = plsc.bitcast(x, jnp.uint32)
    lsb = (u >> 16) & jnp.uint32(1)
    r = (u + jnp.uint32(0x7FFF) + lsb) & jnp.uint32(0xFFFF0000)
    return plsc.bitcast(r, jnp.float32)


def _phase1_body(g_hbm, c2_hbm, src_hbm, dst_hbm, w0b_hbm, kc_hbm,
                 x_hbm, st_hbm,
                 sidx, didx, gv0, gv1, xv0, xv1, stv, wv, kcv,
                 semA, semB, semX):
    w = _wid()
    base0 = w * P1_EDGES
    zero = jnp.zeros((16,), jnp.float32)
    iota = lax.iota(jnp.int32, 16)
    for k in range(8):
        stv[0, pl.ds(k * 16, 16)] = zero
    pltpu.sync_copy(w0b_hbm, wv)
    wvals = [[wv[i, pl.ds(k * 16, 16)] for k in range(4)] for i in range(3)]
    pltpu.sync_copy(src_hbm.at[pl.ds(base0, P1_EDGES)], sidx)
    pltpu.sync_copy(dst_hbm.at[pl.ds(base0, P1_EDGES)], didx)
    pltpu.sync_copy(kc_hbm, kcv)

    def start(c, gv, sem):
        pltpu.async_copy(g_hbm.at[sidx.at[pl.ds(c * P1_CHUNK, P1_CHUNK)]],
                         gv, sem)

    def drain(gv, sem):
        pltpu.make_async_copy(g_hbm.at[pl.ds(0, P1_CHUNK)], gv, sem).wait()

    def compute(c, gv, xv, first_writes):
        def edge_body(e, carry):
            out = list(carry)
            d16 = didx[pl.ds(c * P1_CHUNK + (e // 16) * 16, 16)]
            d_b = _vreg_gather(d16, jnp.full((16,), 0, jnp.int32) + (e % 16))
            kc = plsc.load_gather(kcv, [d_b * 3 + iota])
            rel = _round_bf16(gv[e, pl.ds(64, 16)] - kc)
            r = [_vreg_gather(rel, jnp.full((16,), i, jnp.int32))
                 for i in range(3)]
            for k in range(4):
                g = gv[e, pl.ds(k * 16, 16)]
                acc = g + r[0] * wvals[0][k]
                acc = acc + r[1] * wvals[1][k]
                acc = acc + r[2] * wvals[2][k]
                v = jnp.maximum(acc, 0.0)
                xv[e, pl.ds(k * 16, 16)] = v
                out[k] = carry[k] + v
                out[4 + k] = carry[4 + k] + v * v
            return tuple(out)

        stats = lax.fori_loop(0, P1_CHUNK, edge_body, (zero,) * 8)
        for k in range(8):
            stv[0, pl.ds(k * 16, 16)] += stats[k]
        @pl.when(jnp.logical_not(first_writes))
        def _():
            pltpu.make_async_copy(
                x_hbm.at[pl.ds(0, P1_CHUNK)], xv, semX).wait()
        pltpu.async_copy(
            xv, x_hbm.at[pl.ds(base0 + c * P1_CHUNK, P1_CHUNK)], semX)

    start(0, gv0, semA)

    def pair_body(jp, _):
        c = 2 * jp
        start(c + 1, gv1, semB)
        drain(gv0, semA)
        compute(c, gv0, xv0, jp == 0)
        start(c + 2, gv0, semA)
        drain(gv1, semB)
        compute(c + 1, gv1, xv1, jp == 0)
        return 0

    lax.fori_loop(0, (P1_NCHUNK - 1) // 2, pair_body, 0)
    drain(gv0, semA)
    compute(P1_NCHUNK - 1, gv0, xv0, False)
    pltpu.make_async_copy(x_hbm.at[pl.ds(0, P1_CHUNK)], xv1, semX).wait()
    pltpu.make_async_copy(x_hbm.at[pl.ds(0, P1_CHUNK)], xv0, semX).wait()
    pltpu.sync_copy(stv, st_hbm.at[w])


KC_PAD = K_PAD * 3 + 16


def _phase1(g, c2, src, dst, w0b_r, kc_flat):
    f = pl.kernel(
        _phase1_body,
        out_type=(
            jax.ShapeDtypeStruct((E, 64), jnp.float32),
            jax.ShapeDtypeStruct((NW, 1, 128), jnp.float32),
        ),
        mesh=_mesh(),
        scratch_types=[
            pltpu.VMEM((P1_EDGES,), jnp.int32),
            pltpu.VMEM((P1_EDGES,), jnp.int32),
            pltpu.VMEM((P1_CHUNK, 128), jnp.float32),
            pltpu.VMEM((P1_CHUNK, 128), jnp.float32),
            pltpu.VMEM((P1_CHUNK, 64), jnp.float32),
            pltpu.VMEM((P1_CHUNK, 64), jnp.float32),
            pltpu.VMEM((1, 128), jnp.float32),
            pltpu.VMEM((3, 64), jnp.float32),
            pltpu.VMEM((KC_PAD,), jnp.float32),
            pltpu.SemaphoreType.DMA,
            pltpu.SemaphoreType.DMA,
            pltpu.SemaphoreType.DMA,
        ],
        compiler_params=pltpu.CompilerParams(needs_layout_passes=False),
    )
    return f(g, c2, src, dst, w0b_r, kc_flat)


def _mlp1_body(x_ref, w1_ref, b1_ref, a0_ref, c0_ref, y_ref, st_ref):
    i = pl.program_id(0)
    xn = x_ref[...] * a0_ref[...] + c0_ref[...]
    h = jnp.maximum(_bdot(xn, w1_ref[...]) + b1_ref[...], 0.0)
    for k in range(SM_FG):
        y_ref[k] = jnp.concatenate(
            [h[e4 * MLP1_ROWS:(e4 + 1) * MLP1_ROWS, k * FW:(k + 1) * FW]
             for e4 in range(4)], axis=1)
    st = jnp.stack([jnp.sum(h, axis=0), jnp.sum(h * h, axis=0)])

    @pl.when(i == 0)
    def _():
        st_ref[...] = st

    @pl.when(i > 0)
    def _():
        st_ref[...] += st


def _mlp1(x, w1, b1, a0, c0):
    blk = MLP1_BLK
    return pl.pallas_call(
        _mlp1_body,
        grid=(NBLK,),
        in_specs=[
            pl.BlockSpec((blk, 64), lambda i: (i, 0)),
            pl.BlockSpec((64, 128), lambda i: (0, 0)),
            pl.BlockSpec((1, 128), lambda i: (0, 0)),
            pl.BlockSpec((1, 64), lambda i: (0, 0)),
            pl.BlockSpec((1, 64), lambda i: (0, 0)),
        ],
        out_specs=[
            pl.BlockSpec((SM_FG, MLP1_ROWS, 128), lambda i: (0, i, 0)),
            pl.BlockSpec((2, 128), lambda i: (0, 0)),
        ],
        out_shape=[
            jax.ShapeDtypeStruct((SM_FG, E // 4, 128), jnp.float32),
            jax.ShapeDtypeStruct((2, 128), jnp.float32),
        ],
    )(x, w1, b1, a0, c0)


def _segmax_body(y4_hbm, dst_hbm, p_hbm, acc,
                 d0a, d0b, d0c, d0d, d1a, d1b, d1c, d1d, yv0, yv1,
                 semA, semB):
    didx0 = [d0a, d0b, d0c, d0d]
    didx1 = [d1a, d1b, d1c, d1d]
    w = _wid()
    fg = w % SM_FG
    eg = w // SM_FG
    extra = NBLK % SM_EG
    b0 = eg * (NBLK // SM_EG) + jnp.minimum(eg, extra)
    nb = (NBLK // SM_EG) + jnp.where(eg < extra, 1, 0)
    nsub = nb * SM_SUBS
    ninf = jnp.full((16,), -jnp.inf, jnp.float32)
    iota = lax.iota(jnp.int32, 16)

    def init_body(i, _):
        acc[pl.ds(i * 16, 16)] = ninf
        return 0

    lax.fori_loop(0, ACC_N // 16, init_body, 0)

    def start(s, didx, yv, sem):
        blk = b0 + s // SM_SUBS
        s4 = s % SM_SUBS
        for e4 in range(4):
            pltpu.async_copy(
                dst_hbm.at[pl.ds(
                    blk * MLP1_BLK + e4 * MLP1_ROWS + s4 * SM_SR, SM_SR)],
                didx[e4], sem)
        pltpu.async_copy(
            y4_hbm.at[fg, pl.ds(blk * MLP1_ROWS + s4 * SM_SR, SM_SR)],
            yv, sem)

    def start_if(s, didx, yv, sem):
        @pl.when(s < nsub)
        def _():
            start(s, didx, yv, sem)

    def drain(didx, yv, sem):
        for e4 in range(4):
            pltpu.make_async_copy(
                dst_hbm.at[pl.ds(0, SM_SR)], didx[e4], sem).wait()
        pltpu.make_async_copy(y4_hbm.at[fg, pl.ds(0, SM_SR)], yv, sem).wait()

    def compute(didx, yv):
        for e4 in range(4):
            c0 = e4 * FW

            def grp_body(jj, _):
                d16 = didx[e4][pl.ds(jj * 16, 16)]
                for l in range(16):
                    d_b = _vreg_gather(d16, jnp.full((16,), l, jnp.int32))
                    idx0 = d_b * FW + iota
                    r = jj * 16 + l
                    y0 = yv[r, pl.ds(c0, 16)]
                    y1 = yv[r, pl.ds(c0 + 16, 16)]
                    a0 = plsc.load_gather(acc, [idx0])
                    a1 = plsc.load_gather(acc, [idx0 + 16])
                    plsc.store_scatter(acc, [idx0], jnp.maximum(a0, y0))
                    plsc.store_scatter(acc, [idx0 + 16], jnp.maximum(a1, y1))
                return 0

            lax.fori_loop(0, SM_SR // 16, grp_body, 0)

    start(0, didx0, yv0, semA)

    def pair_body(jp, _):
        s = 2 * jp
        start(s + 1, didx1, yv1, semB)
        drain(didx0, yv0, semA)
        compute(didx0, yv0)
        start_if(s + 2, didx0, yv0, semA)
        drain(didx1, yv1, semB)
        compute(didx1, yv1)
        return 0

    lax.fori_loop(0, nsub // 2, pair_body, 0)
    pltpu.sync_copy(acc, p_hbm.at[pl.ds(w * ACC_N, ACC_N)])


def _segmax(y4, dst):
    f = pl.kernel(
        _segmax_body,
        out_type=jax.ShapeDtypeStruct((NW * ACC_N,), jnp.float32),
        mesh=_mesh(),
        scratch_types=[
            pltpu.VMEM((ACC_N,), jnp.float32),
            pltpu.VMEM((SM_SR,), jnp.int32),
            pltpu.VMEM((SM_SR,), jnp.int32),
            pltpu.VMEM((SM_SR,), jnp.int32),
            pltpu.VMEM((SM_SR,), jnp.int32),
            pltpu.VMEM((SM_SR,), jnp.int32),
            pltpu.VMEM((SM_SR,), jnp.int32),
            pltpu.VMEM((SM_SR,), jnp.int32),
            pltpu.VMEM((SM_SR,), jnp.int32),
            pltpu.VMEM((SM_SR, 128), jnp.float32),
            pltpu.VMEM((SM_SR, 128), jnp.float32),
            pltpu.SemaphoreType.DMA,
            pltpu.SemaphoreType.DMA,
        ],
        compiler_params=pltpu.CompilerParams(needs_layout_passes=False),
    )
    return f(y4, dst)


def _mlp2_body(p_ref, a1_ref, c1_ref, w0_ref, b0_ref, g0_ref, bt0_ref,
               w1_ref, b1_ref, g1_ref, bt1_ref, out_ref):
    rowmask = lax.broadcasted_iota(jnp.int32, (K_PAD, 128), 0) < K_CENTERS
    n = jnp.float32(K_CENTERS)
    agg = jnp.concatenate(
        [jnp.max(p_ref[:, k], axis=0) for k in range(SM_FG)], axis=-1)
    aggn = agg * a1_ref[...] + c1_ref[...]
    t = jnp.maximum(_bdot(aggn, w0_ref[...]) + b0_ref[...], 0.0)
    m = jnp.sum(jnp.where(rowmask, t, 0.0), axis=0, keepdims=True) / n
    v = jnp.sum(jnp.where(rowmask, (t - m) ** 2, 0.0), axis=0, keepdims=True) / n
    tn = (t - m) / jnp.sqrt(v + EPS) * g0_ref[...] + bt0_ref[...]
    u = jnp.maximum(_bdot(tn, w1_ref[...]) + b1_ref[...], 0.0)
    m2 = jnp.sum(jnp.where(rowmask, u, 0.0), axis=0, keepdims=True) / n
    v2 = jnp.sum(jnp.where(rowmask, (u - m2) ** 2, 0.0), axis=0, keepdims=True) / n
    out_ref[...] = (u - m2) / jnp.sqrt(v2 + EPS) * g1_ref[...] + bt1_ref[...]


def _mlp2(p, a1, c1, w0, b0, g0, bt0, w1, b1, g1, bt1):
    full = lambda *s: pl.BlockSpec(s, lambda: tuple(0 for _ in s))
    return pl.pallas_call(
        _mlp2_body,
        in_specs=[
            full(SM_EG, SM_FG, K_PAD, FW),
            full(1, 128), full(1, 128),
            full(128, 128), full(1, 128), full(1, 128), full(1, 128),
            full(128, 128), full(1, 128), full(1, 128), full(1, 128),
        ],
        out_specs=full(K_PAD, 128),
        out_shape=jax.ShapeDtypeStruct((K_PAD, 128), jnp.float32),
    )(p, a1, c1, w0, b0, g0, bt0, w1, b1, g1, bt1)


def kernel(point_features, point_coordinates, keypoint_indices, set_indices,
           pt_W0, pt_b0, pt_g0, pt_bt0, pt_W1, pt_b1, pt_g1, pt_bt1,
           out_W0, out_b0, out_g0, out_bt0, out_W1, out_b1, out_g1, out_bt1):
    src = set_indices[:, 0]
    dst = set_indices[:, 1]
    kidx = keypoint_indices[:, 0]
    kidx_pad = jnp.concatenate(
        [kidx, jnp.broadcast_to(kidx[-1:], (K_PAD - K_CENTERS,))])
    w0a = pt_W0[:D_FEAT]
    w0b_r = pt_W0[D_FEAT:].astype(jnp.bfloat16).astype(jnp.float32)

    t_tab = _prep(point_features, point_coordinates, w0a,
                  pt_b0.reshape(1, 64))
    c2 = _gatherc(t_tab, kidx_pad)
    kc_flat = jnp.concatenate(
        [c2[:, 64:67].reshape(-1), jnp.zeros((16,), jnp.float32)])
    x, st0 = _phase1(t_tab, c2, src, dst, w0b_r, kc_flat)

    s0 = jnp.sum(st0[:, 0, :], axis=0)
    m0 = s0[:64] / E
    v0 = s0[64:] / E - m0 * m0
    a0 = pt_g0 / jnp.sqrt(v0 + EPS)
    c0 = pt_bt0 - m0 * a0

    y4, st1 = _mlp1(x, pt_W1, pt_b1.reshape(1, 128),
                    a0.reshape(1, 64), c0.reshape(1, 64))

    m1 = st1[0] / E
    v1 = st1[1] / E - m1 * m1
    a1 = pt_g1 / jnp.sqrt(v1 + EPS)
    c1 = pt_bt1 - m1 * a1

    p_flat = _segmax(y4, dst)
    p = p_flat.reshape(SM_EG, SM_FG, K_CENTERS, FW)
    p = jnp.pad(p, ((0, 0), (0, 0), (0, K_PAD - K_CENTERS), (0, 0)))
    out = _mlp2(p, a1.reshape(1, 128), c1.reshape(1, 128),
                out_W0, out_b0.reshape(1, 128), out_g0.reshape(1, 128),
                out_bt0.reshape(1, 128),
                out_W1, out_b1.reshape(1, 128), out_g1.reshape(1, 128),
                out_bt1.reshape(1, 128))
    return out[:K_CENTERS]


# speedup vs baseline: 1.2642x; 1.2642x over previous
import functools

import jax
import jax.numpy as jnp
from jax import lax
from jax.experimental import pallas as pl
from jax.experimental.pallas import tpu as pltpu
from jax.experimental.pallas import tpu_sc as plsc

N = 10000
D_FEAT = 128
K_CENTERS = 2500
E = 320000
EPS = 1e-3

NC = 2
NS = 16
NW = NC * NS

P1_EDGES = E // NW
P1_CHUNK = 80
P1_NCHUNK = P1_EDGES // P1_CHUNK

MLP1_BLK = 2560
MLP1_ROWS = MLP1_BLK // 4
NBLK = E // MLP1_BLK
SM_SR = 160
SM_SUBS = MLP1_ROWS // SM_SR

SM_FG = 4
SM_EG = NW // SM_FG
FW = D_FEAT // SM_FG
ACC_N = K_CENTERS * FW

K_PAD = 2560


def _mesh():
    return plsc.VectorSubcoreMesh(core_axis_name="c", subcore_axis_name="s")


def _wid():
    return lax.axis_index("s") * NC + lax.axis_index("c")


def _vreg_gather(vec, idx):
    dn = lax.GatherDimensionNumbers(
        offset_dims=(), collapsed_slice_dims=(0,), start_index_map=(0,))
    return lax.gather(vec, idx[:, None], dn, slice_sizes=(1,),
                      mode=lax.GatherScatterMode.PROMISE_IN_BOUNDS)


def _bdot(a, b):
    return jnp.dot(a.astype(jnp.bfloat16), b.astype(jnp.bfloat16),
                   preferred_element_type=jnp.float32)


def _prep_body(pf_ref, pc_ref, w0a_ref, b0_ref, t_ref):
    blk = pf_ref.shape[0]
    g = _bdot(pf_ref[...], w0a_ref[...]) + b0_ref[...]
    t_ref[...] = jnp.concatenate(
        [g, pc_ref[...], jnp.zeros((blk, 61), jnp.float32)], axis=-1)


def _prep(pf, pc, w0a, b0):
    blk = 2000
    grid = N // blk
    return pl.pallas_call(
        _prep_body,
        grid=(grid,),
        in_specs=[
            pl.BlockSpec((blk, D_FEAT), lambda i: (i, 0)),
            pl.BlockSpec((blk, 3), lambda i: (i, 0)),
            pl.BlockSpec((D_FEAT, 64), lambda i: (0, 0)),
            pl.BlockSpec((1, 64), lambda i: (0, 0)),
        ],
        out_specs=pl.BlockSpec((blk, 128), lambda i: (i, 0)),
        out_shape=jax.ShapeDtypeStruct((N, 128), jnp.float32),
    )(pf, pc, w0a, b0)


def _gatherc_body(d_hbm, kidx_hbm, c2_hbm, idx_v, rows_v, sem):
    w = _wid()
    base = w * (K_PAD // NW)
    pltpu.sync_copy(kidx_hbm.at[pl.ds(base, K_PAD // NW)], idx_v)
    pltpu.async_copy(d_hbm.at[idx_v], rows_v, sem).wait()
    pltpu.sync_copy(rows_v, c2_hbm.at[pl.ds(base, K_PAD // NW)])


def _gatherc(d, kidx_pad):
    per = K_PAD // NW
    f = pl.kernel(
        _gatherc_body,
        out_type=jax.ShapeDtypeStruct((K_PAD, 128), jnp.float32),
        mesh=_mesh(),
        scratch_types=[
            pltpu.VMEM((per,), jnp.int32),
            pltpu.VMEM((per, 128), jnp.float32),
            pltpu.SemaphoreType.DMA,
        ],
    )
    return f(d, kidx_pad)


def _round_bf16(x):
    u = plsc.bitcast(x, jnp.uint32)
    lsb = (u >> 16) & jnp.uint32(1)
    r = (u + jnp.uint32(0x7FFF) + lsb) & jnp.uint32(0xFFFF0000)
    return plsc.bitcast(r, jnp.float32)


def _phase1_body(g_hbm, c2_hbm, src_hbm, dst_hbm, w0b_hbm, x_hbm, st_hbm,
                 sidx, didx, gv0, cv0, gv1, cv1, xv0, xv1, stv, wv,
                 semA, semB, semX):
    w = _wid()
    base0 = w * P1_EDGES
    zero = jnp.zeros((16,), jnp.float32)
    for k in range(8):
        stv[0, pl.ds(k * 16, 16)] = zero
    pltpu.sync_copy(w0b_hbm, wv)
    wvals = [[wv[i, pl.ds(k * 16, 16)] for k in range(4)] for i in range(3)]
    pltpu.sync_copy(src_hbm.at[pl.ds(base0, P1_EDGES)], sidx)
    pltpu.sync_copy(dst_hbm.at[pl.ds(base0, P1_EDGES)], didx)

    def start(c, gv, cv, sem):
        pltpu.async_copy(g_hbm.at[sidx.at[pl.ds(c * P1_CHUNK, P1_CHUNK)]],
                         gv, sem)
        pltpu.async_copy(c2_hbm.at[didx.at[pl.ds(c * P1_CHUNK, P1_CHUNK)]],
                         cv, sem)

    def drain(gv, cv, sem):
        pltpu.make_async_copy(g_hbm.at[pl.ds(0, P1_CHUNK)], gv, sem).wait()
        pltpu.make_async_copy(c2_hbm.at[pl.ds(0, P1_CHUNK)], cv, sem).wait()

    def compute(c, gv, cv, xv, first_writes):
        def edge_body(e, carry):
            out = list(carry)
            rel = _round_bf16(gv[e, pl.ds(64, 16)] - cv[e, pl.ds(64, 16)])
            r = [_vreg_gather(rel, jnp.full((16,), i, jnp.int32))
                 for i in range(3)]
            for k in range(4):
                g = gv[e, pl.ds(k * 16, 16)]
                acc = g + r[0] * wvals[0][k]
                acc = acc + r[1] * wvals[1][k]
                acc = acc + r[2] * wvals[2][k]
                v = jnp.maximum(acc, 0.0)
                xv[e, pl.ds(k * 16, 16)] = v
                out[k] = carry[k] + v
                out[4 + k] = carry[4 + k] + v * v
            return tuple(out)

        stats = lax.fori_loop(0, P1_CHUNK, edge_body, (zero,) * 8)
        for k in range(8):
            stv[0, pl.ds(k * 16, 16)] += stats[k]
        @pl.when(jnp.logical_not(first_writes))
        def _():
            pltpu.make_async_copy(
                x_hbm.at[pl.ds(0, P1_CHUNK)], xv, semX).wait()
        pltpu.async_copy(
            xv, x_hbm.at[pl.ds(base0 + c * P1_CHUNK, P1_CHUNK)], semX)

    start(0, gv0, cv0, semA)

    def pair_body(jp, _):
        c = 2 * jp
        start(c + 1, gv1, cv1, semB)
        drain(gv0, cv0, semA)
        compute(c, gv0, cv0, xv0, jp == 0)
        start(c + 2, gv0, cv0, semA)
        drain(gv1, cv1, semB)
        compute(c + 1, gv1, cv1, xv1, jp == 0)
        return 0

    lax.fori_loop(0, (P1_NCHUNK - 1) // 2, pair_body, 0)
    drain(gv0, cv0, semA)
    compute(P1_NCHUNK - 1, gv0, cv0, xv0, False)
    pltpu.make_async_copy(x_hbm.at[pl.ds(0, P1_CHUNK)], xv1, semX).wait()
    pltpu.make_async_copy(x_hbm.at[pl.ds(0, P1_CHUNK)], xv0, semX).wait()
    pltpu.sync_copy(stv, st_hbm.at[w])


def _phase1(g, c2, src, dst, w0b_r):
    f = pl.kernel(
        _phase1_body,
        out_type=(
            jax.ShapeDtypeStruct((E, 64), jnp.float32),
            jax.ShapeDtypeStruct((NW, 1, 128), jnp.float32),
        ),
        mesh=_mesh(),
        scratch_types=[
            pltpu.VMEM((P1_EDGES,), jnp.int32),
            pltpu.VMEM((P1_EDGES,), jnp.int32),
            pltpu.VMEM((P1_CHUNK, 128), jnp.float32),
            pltpu.VMEM((P1_CHUNK, 128), jnp.float32),
            pltpu.VMEM((P1_CHUNK, 128), jnp.float32),
            pltpu.VMEM((P1_CHUNK, 128), jnp.float32),
            pltpu.VMEM((P1_CHUNK, 64), jnp.float32),
            pltpu.VMEM((P1_CHUNK, 64), jnp.float32),
            pltpu.VMEM((1, 128), jnp.float32),
            pltpu.VMEM((3, 64), jnp.float32),
            pltpu.SemaphoreType.DMA,
            pltpu.SemaphoreType.DMA,
            pltpu.SemaphoreType.DMA,
        ],
        compiler_params=pltpu.CompilerParams(needs_layout_passes=False),
    )
    return f(g, c2, src, dst, w0b_r)


def _mlp1_body(x_ref, w1_ref, b1_ref, st0_ref, g0_ref, bt0_ref,
               y_ref, st_ref):
    i = pl.program_id(0)
    s0 = jnp.sum(st0_ref[...], axis=0)
    m0 = s0[:, :64] / E
    v0 = s0[:, 64:] / E - m0 * m0
    a0 = g0_ref[...] / jnp.sqrt(v0 + EPS)
    c0 = bt0_ref[...] - m0 * a0
    xn = x_ref[...] * a0 + c0
    h = jnp.maximum(_bdot(xn, w1_ref[...]) + b1_ref[...], 0.0)
    for k in range(SM_FG):
        y_ref[k] = jnp.concatenate(
            [h[e4 * MLP1_ROWS:(e4 + 1) * MLP1_ROWS, k * FW:(k + 1) * FW]
             for e4 in range(4)], axis=1)
    st = jnp.stack([jnp.sum(h, axis=0), jnp.sum(h * h, axis=0)])

    @pl.when(i == 0)
    def _():
        st_ref[...] = st

    @pl.when(i > 0)
    def _():
        st_ref[...] += st


def _mlp1(x, w1, b1, st0, g0, bt0):
    blk = MLP1_BLK
    return pl.pallas_call(
        _mlp1_body,
        grid=(NBLK,),
        in_specs=[
            pl.BlockSpec((blk, 64), lambda i: (i, 0)),
            pl.BlockSpec((64, 128), lambda i: (0, 0)),
            pl.BlockSpec((1, 128), lambda i: (0, 0)),
            pl.BlockSpec((NW, 1, 128), lambda i: (0, 0, 0)),
            pl.BlockSpec((1, 64), lambda i: (0, 0)),
            pl.BlockSpec((1, 64), lambda i: (0, 0)),
        ],
        out_specs=[
            pl.BlockSpec((SM_FG, MLP1_ROWS, 128), lambda i: (0, i, 0)),
            pl.BlockSpec((2, 128), lambda i: (0, 0)),
        ],
        out_shape=[
            jax.ShapeDtypeStruct((SM_FG, E // 4, 128), jnp.float32),
            jax.ShapeDtypeStruct((2, 128), jnp.float32),
        ],
    )(x, w1, b1, st0, g0, bt0)


def _segmax_body(y4_hbm, dst_hbm, p_hbm, acc,
                 d0a, d0b, d0c, d0d, d1a, d1b, d1c, d1d, yv0, yv1,
                 semA, semB):
    didx0 = [d0a, d0b, d0c, d0d]
    didx1 = [d1a, d1b, d1c, d1d]
    w = _wid()
    fg = w % SM_FG
    eg = w // SM_FG
    extra = NBLK % SM_EG
    b0 = eg * (NBLK // SM_EG) + jnp.minimum(eg, extra)
    nb = (NBLK // SM_EG) + jnp.where(eg < extra, 1, 0)
    nsub = nb * SM_SUBS
    ninf = jnp.full((16,), -jnp.inf, jnp.float32)
    iota = lax.iota(jnp.int32, 16)

    def init_body(i, _):
        acc[pl.ds(i * 16, 16)] = ninf
        return 0

    lax.fori_loop(0, ACC_N // 16, init_body, 0)

    def start(s, didx, yv, sem):
        blk = b0 + s // SM_SUBS
        s4 = s % SM_SUBS
        for e4 in range(4):
            pltpu.async_copy(
                dst_hbm.at[pl.ds(
                    blk * MLP1_BLK + e4 * MLP1_ROWS + s4 * SM_SR, SM_SR)],
                didx[e4], sem)
        pltpu.async_copy(
            y4_hbm.at[fg, pl.ds(blk * MLP1_ROWS + s4 * SM_SR, SM_SR)],
            yv, sem)

    def start_if(s, didx, yv, sem):
        @pl.when(s < nsub)
        def _():
            start(s, didx, yv, sem)

    def drain(didx, yv, sem):
        for e4 in range(4):
            pltpu.make_async_copy(
                dst_hbm.at[pl.ds(0, SM_SR)], didx[e4], sem).wait()
        pltpu.make_async_copy(y4_hbm.at[fg, pl.ds(0, SM_SR)], yv, sem).wait()

    def compute(didx, yv):
        for e4 in range(4):
            c0 = e4 * FW

            def grp_body(jj, _):
                d16 = didx[e4][pl.ds(jj * 16, 16)]
                for l in range(16):
                    d_b = _vreg_gather(d16, jnp.full((16,), l, jnp.int32))
                    idx0 = d_b * FW + iota
                    r = jj * 16 + l
                    y0 = yv[r, pl.ds(c0, 16)]
                    y1 = yv[r, pl.ds(c0 + 16, 16)]
                    a0 = plsc.load_gather(acc, [idx0])
                    a1 = plsc.load_gather(acc, [idx0 + 16])
                    plsc.store_scatter(acc, [idx0], jnp.maximum(a0, y0))
                    plsc.store_scatter(acc, [idx0 + 16], jnp.maximum(a1, y1))
                return 0

            lax.fori_loop(0, SM_SR // 16, grp_body, 0)

    start(0, didx0, yv0, semA)

    def pair_body(jp, _):
        s = 2 * jp
        start(s + 1, didx1, yv1, semB)
        drain(didx0, yv0, semA)
        compute(didx0, yv0)
        start_if(s + 2, didx0, yv0, semA)
        drain(didx1, yv1, semB)
        compute(didx1, yv1)
        return 0

    lax.fori_loop(0, nsub // 2, pair_body, 0)
    pltpu.sync_copy(acc, p_hbm.at[pl.ds(w * ACC_N, ACC_N)])


def _segmax(y4, dst):
    f = pl.kernel(
        _segmax_body,
        out_type=jax.ShapeDtypeStruct((NW * ACC_N,), jnp.float32),
        mesh=_mesh(),
        scratch_types=[
            pltpu.VMEM((ACC_N,), jnp.float32),
            pltpu.VMEM((SM_SR,), jnp.int32),
            pltpu.VMEM((SM_SR,), jnp.int32),
            pltpu.VMEM((SM_SR,), jnp.int32),
            pltpu.VMEM((SM_SR,), jnp.int32),
            pltpu.VMEM((SM_SR,), jnp.int32),
            pltpu.VMEM((SM_SR,), jnp.int32),
            pltpu.VMEM((SM_SR,), jnp.int32),
            pltpu.VMEM((SM_SR,), jnp.int32),
            pltpu.VMEM((SM_SR, 128), jnp.float32),
            pltpu.VMEM((SM_SR, 128), jnp.float32),
            pltpu.SemaphoreType.DMA,
            pltpu.SemaphoreType.DMA,
        ],
        compiler_params=pltpu.CompilerParams(needs_layout_passes=False),
    )
    return f(y4, dst)


def _mlp2_body(p_ref, st1_ref, pg1_ref, pbt1_ref,
               w0_ref, b0_ref, g0_ref, bt0_ref,
               w1_ref, b1_ref, g1_ref, bt1_ref, out_ref):
    rowmask = lax.broadcasted_iota(jnp.int32, (K_PAD, 128), 0) < K_CENTERS
    n = jnp.float32(K_CENTERS)
    st1 = st1_ref[...]
    m1 = st1[:1] / E
    v1 = st1[1:] / E - m1 * m1
    a1 = pg1_ref[...] / jnp.sqrt(v1 + EPS)
    c1 = pbt1_ref[...] - m1 * a1
    agg = jnp.concatenate(
        [jnp.max(p_ref[:, k], axis=0) for k in range(SM_FG)], axis=-1)
    aggn = agg * a1 + c1
    t = jnp.maximum(_bdot(aggn, w0_ref[...]) + b0_ref[...], 0.0)
    m = jnp.sum(jnp.where(rowmask, t, 0.0), axis=0, keepdims=True) / n
    v = jnp.sum(jnp.where(rowmask, (t - m) ** 2, 0.0), axis=0, keepdims=True) / n
    tn = (t - m) / jnp.sqrt(v + EPS) * g0_ref[...] + bt0_ref[...]
    u = jnp.maximum(_bdot(tn, w1_ref[...]) + b1_ref[...], 0.0)
    m2 = jnp.sum(jnp.where(rowmask, u, 0.0), axis=0, keepdims=True) / n
    v2 = jnp.sum(jnp.where(rowmask, (u - m2) ** 2, 0.0), axis=0, keepdims=True) / n
    out_ref[...] = (u - m2) / jnp.sqrt(v2 + EPS) * g1_ref[...] + bt1_ref[...]


def _mlp2(p, st1, pg1, pbt1, w0, b0, g0, bt0, w1, b1, g1, bt1):
    full = lambda *s: pl.BlockSpec(s, lambda: tuple(0 for _ in s))
    return pl.pallas_call(
        _mlp2_body,
        in_specs=[
            full(SM_EG, SM_FG, K_PAD, FW),
            full(2, 128), full(1, 128), full(1, 128),
            full(128, 128), full(1, 128), full(1, 128), full(1, 128),
            full(128, 128), full(1, 128), full(1, 128), full(1, 128),
        ],
        out_specs=full(K_PAD, 128),
        out_shape=jax.ShapeDtypeStruct((K_PAD, 128), jnp.float32),
    )(p, st1, pg1, pbt1, w0, b0, g0, bt0, w1, b1, g1, bt1)


def kernel(point_features, point_coordinates, keypoint_indices, set_indices,
           pt_W0, pt_b0, pt_g0, pt_bt0, pt_W1, pt_b1, pt_g1, pt_bt1,
           out_W0, out_b0, out_g0, out_bt0, out_W1, out_b1, out_g1, out_bt1):
    src = set_indices[:, 0]
    dst = set_indices[:, 1]
    kidx = keypoint_indices[:, 0]
    kidx_pad = jnp.concatenate(
        [kidx, jnp.broadcast_to(kidx[-1:], (K_PAD - K_CENTERS,))])
    w0a = pt_W0[:D_FEAT]
    w0b_r = pt_W0[D_FEAT:].astype(jnp.bfloat16).astype(jnp.float32)

    t_tab = _prep(point_features, point_coordinates, w0a,
                  pt_b0.reshape(1, 64))
    c2 = _gatherc(t_tab, kidx_pad)
    x, st0 = _phase1(t_tab, c2, src, dst, w0b_r)

    y4, st1 = _mlp1(x, pt_W1, pt_b1.reshape(1, 128), st0,
                    pt_g0.reshape(1, 64), pt_bt0.reshape(1, 64))

    p_flat = _segmax(y4, dst)
    p = p_flat.reshape(SM_EG, SM_FG, K_CENTERS, FW)
    p = jnp.pad(p, ((0, 0), (0, 0), (0, K_PAD - K_CENTERS), (0, 0)))
    out = _mlp2(p, st1, pt_g1.reshape(1, 128), pt_bt1.reshape(1, 128),
                out_W0, out_b0.reshape(1, 128), out_g0.reshape(1, 128),
                out_bt0.reshape(1, 128),
                out_W1, out_b1.reshape(1, 128), out_g1.reshape(1, 128),
                out_bt1.reshape(1, 128))
    return out[:K_CENTERS]


# speedup vs baseline: 1.3025x; 1.0303x over previous
import functools

import jax
import jax.numpy as jnp
from jax import lax
from jax.experimental import pallas as pl
from jax.experimental.pallas import tpu as pltpu
from jax.experimental.pallas import tpu_sc as plsc

N = 10000
D_FEAT = 128
K_CENTERS = 2500
E = 320000
EPS = 1e-3

NC = 2
NS = 16
NW = NC * NS

P1_EDGES = E // NW
P1_CHUNK = 80
P1_NCHUNK = P1_EDGES // P1_CHUNK

MLP1_BLK = 2560
MLP1_ROWS = MLP1_BLK // 4
NBLK = E // MLP1_BLK
SM_SR = 160
SM_SUBS = MLP1_ROWS // SM_SR

SM_FG = 4
SM_EG = NW // SM_FG
FW = D_FEAT // SM_FG
ACC_N = K_CENTERS * FW

K_PAD = 2560


def _mesh():
    return plsc.VectorSubcoreMesh(core_axis_name="c", subcore_axis_name="s")


def _wid():
    return lax.axis_index("s") * NC + lax.axis_index("c")


def _vreg_gather(vec, idx):
    dn = lax.GatherDimensionNumbers(
        offset_dims=(), collapsed_slice_dims=(0,), start_index_map=(0,))
    return lax.gather(vec, idx[:, None], dn, slice_sizes=(1,),
                      mode=lax.GatherScatterMode.PROMISE_IN_BOUNDS)


def _bdot(a, b):
    return jnp.dot(a.astype(jnp.bfloat16), b.astype(jnp.bfloat16),
                   preferred_element_type=jnp.float32)


def _prep_body(pf_ref, pc_ref, w0a_ref, b0_ref, t_ref):
    blk = pf_ref.shape[0]
    g = _bdot(pf_ref[...], w0a_ref[...]) + b0_ref[...]
    t_ref[...] = jnp.concatenate(
        [g, pc_ref[...], jnp.zeros((blk, 61), jnp.float32)], axis=-1)


def _prep(pf, pc, w0a, b0):
    blk = 2000
    grid = N // blk
    return pl.pallas_call(
        _prep_body,
        grid=(grid,),
        in_specs=[
            pl.BlockSpec((blk, D_FEAT), lambda i: (i, 0)),
            pl.BlockSpec((blk, 3), lambda i: (i, 0)),
            pl.BlockSpec((D_FEAT, 64), lambda i: (0, 0)),
            pl.BlockSpec((1, 64), lambda i: (0, 0)),
        ],
        out_specs=pl.BlockSpec((blk, 128), lambda i: (i, 0)),
        out_shape=jax.ShapeDtypeStruct((N, 128), jnp.float32),
    )(pf, pc, w0a, b0)


def _gatherc_body(d_hbm, kidx_hbm, c2_hbm, idx_v, rows_v, sem):
    w = _wid()
    base = w * (K_PAD // NW)
    pltpu.sync_copy(kidx_hbm.at[pl.ds(base, K_PAD // NW)], idx_v)
    pltpu.async_copy(d_hbm.at[idx_v], rows_v, sem).wait()
    pltpu.sync_copy(rows_v, c2_hbm.at[pl.ds(base, K_PAD // NW)])


def _gatherc(d, kidx_pad):
    per = K_PAD // NW
    f = pl.kernel(
        _gatherc_body,
        out_type=jax.ShapeDtypeStruct((K_PAD, 128), jnp.float32),
        mesh=_mesh(),
        scratch_types=[
            pltpu.VMEM((per,), jnp.int32),
            pltpu.VMEM((per, 128), jnp.float32),
            pltpu.SemaphoreType.DMA,
        ],
    )
    return f(d, kidx_pad)


def _round_bf16(x):
    u = plsc.bitcast(x, jnp.uint32)
    lsb = (u >> 16) & jnp.uint32(1)
    r = (u + jnp.uint32(0x7FFF) + lsb) & jnp.uint32(0xFFFF0000)
    return plsc.bitcast(r, jnp.float32)


def _phase1_body(g_hbm, c2_hbm, src_hbm, dst_hbm, w0b_hbm, x_hbm, st_hbm,
                 sidx, didx, gv0, cv0, gv1, cv1, xv0, xv1, stv, wv,
                 semA, semB, semX):
    w = _wid()
    base0 = w * P1_EDGES
    zero = jnp.zeros((16,), jnp.float32)
    for k in range(8):
        stv[0, pl.ds(k * 16, 16)] = zero
    pltpu.sync_copy(w0b_hbm, wv)
    wvals = [[wv[i, pl.ds(k * 16, 16)] for k in range(4)] for i in range(3)]
    pltpu.sync_copy(src_hbm.at[pl.ds(base0, P1_EDGES)], sidx)
    pltpu.sync_copy(dst_hbm.at[pl.ds(base0, P1_EDGES)], didx)

    def start(c, gv, cv, sem):
        pltpu.async_copy(g_hbm.at[sidx.at[pl.ds(c * P1_CHUNK, P1_CHUNK)]],
                         gv, sem)
        pltpu.async_copy(c2_hbm.at[didx.at[pl.ds(c * P1_CHUNK, P1_CHUNK)]],
                         cv, sem)

    def drain(gv, cv, sem):
        pltpu.make_async_copy(g_hbm.at[pl.ds(0, P1_CHUNK)], gv, sem).wait()
        pltpu.make_async_copy(c2_hbm.at[pl.ds(0, P1_CHUNK)], cv, sem).wait()

    def compute(c, gv, cv, xv, first_writes):
        def edge_body(e, carry):
            out = list(carry)
            rel = _round_bf16(gv[e, pl.ds(64, 16)] - cv[e, pl.ds(64, 16)])
            r = [_vreg_gather(rel, jnp.full((16,), i, jnp.int32))
                 for i in range(3)]
            for k in range(4):
                g = gv[e, pl.ds(k * 16, 16)]
                acc = g + r[0] * wvals[0][k]
                acc = acc + r[1] * wvals[1][k]
                acc = acc + r[2] * wvals[2][k]
                v = jnp.maximum(acc, 0.0)
                xv[e, pl.ds(k * 16, 16)] = v
                out[k] = carry[k] + v
                out[4 + k] = carry[4 + k] + v * v
            return tuple(out)

        stats = lax.fori_loop(0, P1_CHUNK, edge_body, (zero,) * 8)
        for k in range(8):
            stv[0, pl.ds(k * 16, 16)] += stats[k]
        @pl.when(jnp.logical_not(first_writes))
        def _():
            pltpu.make_async_copy(
                x_hbm.at[pl.ds(0, P1_CHUNK)], xv, semX).wait()
        pltpu.async_copy(
            xv, x_hbm.at[pl.ds(base0 + c * P1_CHUNK, P1_CHUNK)], semX)

    start(0, gv0, cv0, semA)

    def pair_body(jp, _):
        c = 2 * jp
        start(c + 1, gv1, cv1, semB)
        drain(gv0, cv0, semA)
        compute(c, gv0, cv0, xv0, jp == 0)
        start(c + 2, gv0, cv0, semA)
        drain(gv1, cv1, semB)
        compute(c + 1, gv1, cv1, xv1, jp == 0)
        return 0

    lax.fori_loop(0, (P1_NCHUNK - 1) // 2, pair_body, 0)
    drain(gv0, cv0, semA)
    compute(P1_NCHUNK - 1, gv0, cv0, xv0, False)
    pltpu.make_async_copy(x_hbm.at[pl.ds(0, P1_CHUNK)], xv1, semX).wait()
    pltpu.make_async_copy(x_hbm.at[pl.ds(0, P1_CHUNK)], xv0, semX).wait()
    pltpu.sync_copy(stv, st_hbm.at[w])


def _phase1(g, c2, src, dst, w0b_r):
    f = pl.kernel(
        _phase1_body,
        out_type=(
            jax.ShapeDtypeStruct((E, 64), jnp.float32),
            jax.ShapeDtypeStruct((NW, 1, 128), jnp.float32),
        ),
        mesh=_mesh(),
        scratch_types=[
            pltpu.VMEM((P1_EDGES,), jnp.int32),
            pltpu.VMEM((P1_EDGES,), jnp.int32),
            pltpu.VMEM((P1_CHUNK, 128), jnp.float32),
            pltpu.VMEM((P1_CHUNK, 128), jnp.float32),
            pltpu.VMEM((P1_CHUNK, 128), jnp.float32),
            pltpu.VMEM((P1_CHUNK, 128), jnp.float32),
            pltpu.VMEM((P1_CHUNK, 64), jnp.float32),
            pltpu.VMEM((P1_CHUNK, 64), jnp.float32),
            pltpu.VMEM((1, 128), jnp.float32),
            pltpu.VMEM((3, 64), jnp.float32),
            pltpu.SemaphoreType.DMA,
            pltpu.SemaphoreType.DMA,
            pltpu.SemaphoreType.DMA,
        ],
        compiler_params=pltpu.CompilerParams(needs_layout_passes=False),
    )
    return f(g, c2, src, dst, w0b_r)


def _mlp1_body(x_ref, w1_ref, b1_ref, a0_ref, c0_ref, y_ref, st_ref):
    i = pl.program_id(0)
    xn = x_ref[...] * a0_ref[...] + c0_ref[...]
    h = jnp.maximum(_bdot(xn, w1_ref[...]) + b1_ref[...], 0.0)
    for k in range(SM_FG):
        y_ref[k] = jnp.concatenate(
            [h[e4 * MLP1_ROWS:(e4 + 1) * MLP1_ROWS, k * FW:(k + 1) * FW]
             for e4 in range(4)], axis=1)
    st = jnp.stack([jnp.sum(h, axis=0), jnp.sum(h * h, axis=0)])

    @pl.when(i == 0)
    def _():
        st_ref[...] = st

    @pl.when(i > 0)
    def _():
        st_ref[...] += st


def _mlp1(x, w1, b1, a0, c0):
    blk = MLP1_BLK
    return pl.pallas_call(
        _mlp1_body,
        grid=(NBLK,),
        in_specs=[
            pl.BlockSpec((blk, 64), lambda i: (i, 0)),
            pl.BlockSpec((64, 128), lambda i: (0, 0)),
            pl.BlockSpec((1, 128), lambda i: (0, 0)),
            pl.BlockSpec((1, 64), lambda i: (0, 0)),
            pl.BlockSpec((1, 64), lambda i: (0, 0)),
        ],
        out_specs=[
            pl.BlockSpec((SM_FG, MLP1_ROWS, 128), lambda i: (0, i, 0)),
            pl.BlockSpec((2, 128), lambda i: (0, 0)),
        ],
        out_shape=[
            jax.ShapeDtypeStruct((SM_FG, E // 4, 128), jnp.float32),
            jax.ShapeDtypeStruct((2, 128), jnp.float32),
        ],
    )(x, w1, b1, a0, c0)


def _segmax_body(y4_hbm, dst_hbm, p_hbm, acc,
                 d0a, d0b, d0c, d0d, d1a, d1b, d1c, d1d, yv0, yv1,
                 semA, semB):
    didx0 = [d0a, d0b, d0c, d0d]
    didx1 = [d1a, d1b, d1c, d1d]
    w = _wid()
    fg = w % SM_FG
    eg = w // SM_FG
    extra = NBLK % SM_EG
    b0 = eg * (NBLK // SM_EG) + jnp.minimum(eg, extra)
    nb = (NBLK // SM_EG) + jnp.where(eg < extra, 1, 0)
    nsub = nb * SM_SUBS
    ninf = jnp.full((16,), -jnp.inf, jnp.float32)
    iota = lax.iota(jnp.int32, 16)

    def init_body(i, _):
        acc[pl.ds(i * 16, 16)] = ninf
        return 0

    lax.fori_loop(0, ACC_N // 16, init_body, 0)

    def start(s, didx, yv, sem):
        blk = b0 + s // SM_SUBS
        s4 = s % SM_SUBS
        for e4 in range(4):
            pltpu.async_copy(
                dst_hbm.at[pl.ds(
                    blk * MLP1_BLK + e4 * MLP1_ROWS + s4 * SM_SR, SM_SR)],
                didx[e4], sem)
        pltpu.async_copy(
            y4_hbm.at[fg, pl.ds(blk * MLP1_ROWS + s4 * SM_SR, SM_SR)],
            yv, sem)

    def start_if(s, didx, yv, sem):
        @pl.when(s < nsub)
        def _():
            start(s, didx, yv, sem)

    def drain(didx, yv, sem):
        for e4 in range(4):
            pltpu.make_async_copy(
                dst_hbm.at[pl.ds(0, SM_SR)], didx[e4], sem).wait()
        pltpu.make_async_copy(y4_hbm.at[fg, pl.ds(0, SM_SR)], yv, sem).wait()

    def compute(didx, yv):
        for e4 in range(4):
            c0 = e4 * FW

            def grp_body(jj, _):
                d16 = didx[e4][pl.ds(jj * 16, 16)]
                for l in range(16):
                    d_b = _vreg_gather(d16, jnp.full((16,), l, jnp.int32))
                    idx0 = d_b * FW + iota
                    r = jj * 16 + l
                    y0 = yv[r, pl.ds(c0, 16)]
                    y1 = yv[r, pl.ds(c0 + 16, 16)]
                    a0 = plsc.load_gather(acc, [idx0])
                    a1 = plsc.load_gather(acc, [idx0 + 16])
                    plsc.store_scatter(acc, [idx0], jnp.maximum(a0, y0))
                    plsc.store_scatter(acc, [idx0 + 16], jnp.maximum(a1, y1))
                return 0

            lax.fori_loop(0, SM_SR // 16, grp_body, 0)

    start(0, didx0, yv0, semA)

    def pair_body(jp, _):
        s = 2 * jp
        start(s + 1, didx1, yv1, semB)
        drain(didx0, yv0, semA)
        compute(didx0, yv0)
        start_if(s + 2, didx0, yv0, semA)
        drain(didx1, yv1, semB)
        compute(didx1, yv1)
        return 0

    lax.fori_loop(0, nsub // 2, pair_body, 0)
    pltpu.sync_copy(acc, p_hbm.at[pl.ds(w * ACC_N, ACC_N)])


def _segmax(y4, dst):
    f = pl.kernel(
        _segmax_body,
        out_type=jax.ShapeDtypeStruct((NW * ACC_N,), jnp.float32),
        mesh=_mesh(),
        scratch_types=[
            pltpu.VMEM((ACC_N,), jnp.float32),
            pltpu.VMEM((SM_SR,), jnp.int32),
            pltpu.VMEM((SM_SR,), jnp.int32),
            pltpu.VMEM((SM_SR,), jnp.int32),
            pltpu.VMEM((SM_SR,), jnp.int32),
            pltpu.VMEM((SM_SR,), jnp.int32),
            pltpu.VMEM((SM_SR,), jnp.int32),
            pltpu.VMEM((SM_SR,), jnp.int32),
            pltpu.VMEM((SM_SR,), jnp.int32),
            pltpu.VMEM((SM_SR, 128), jnp.float32),
            pltpu.VMEM((SM_SR, 128), jnp.float32),
            pltpu.SemaphoreType.DMA,
            pltpu.SemaphoreType.DMA,
        ],
        compiler_params=pltpu.CompilerParams(needs_layout_passes=False),
    )
    return f(y4, dst)


def _mlp2_body(p_ref, a1_ref, c1_ref, w0_ref, b0_ref, g0_ref, bt0_ref,
               w1_ref, b1_ref, g1_ref, bt1_ref, out_ref):
    rowmask = lax.broadcasted_iota(jnp.int32, (K_PAD, 128), 0) < K_CENTERS
    n = jnp.float32(K_CENTERS)
    agg = jnp.concatenate(
        [jnp.max(p_ref[:, k], axis=0) for k in range(SM_FG)], axis=-1)
    aggn = agg * a1_ref[...] + c1_ref[...]
    t = jnp.maximum(_bdot(aggn, w0_ref[...]) + b0_ref[...], 0.0)
    m = jnp.sum(jnp.where(rowmask, t, 0.0), axis=0, keepdims=True) / n
    v = jnp.sum(jnp.where(rowmask, (t - m) ** 2, 0.0), axis=0, keepdims=True) / n
    tn = (t - m) / jnp.sqrt(v + EPS) * g0_ref[...] + bt0_ref[...]
    u = jnp.maximum(_bdot(tn, w1_ref[...]) + b1_ref[...], 0.0)
    m2 = jnp.sum(jnp.where(rowmask, u, 0.0), axis=0, keepdims=True) / n
    v2 = jnp.sum(jnp.where(rowmask, (u - m2) ** 2, 0.0), axis=0, keepdims=True) / n
    out_ref[...] = (u - m2) / jnp.sqrt(v2 + EPS) * g1_ref[...] + bt1_ref[...]


def _mlp2(p, a1, c1, w0, b0, g0, bt0, w1, b1, g1, bt1):
    full = lambda *s: pl.BlockSpec(s, lambda: tuple(0 for _ in s))
    return pl.pallas_call(
        _mlp2_body,
        in_specs=[
            full(SM_EG, SM_FG, K_PAD, FW),
            full(1, 128), full(1, 128),
            full(128, 128), full(1, 128), full(1, 128), full(1, 128),
            full(128, 128), full(1, 128), full(1, 128), full(1, 128),
        ],
        out_specs=full(K_PAD, 128),
        out_shape=jax.ShapeDtypeStruct((K_PAD, 128), jnp.float32),
    )(p, a1, c1, w0, b0, g0, bt0, w1, b1, g1, bt1)


def kernel(point_features, point_coordinates, keypoint_indices, set_indices,
           pt_W0, pt_b0, pt_g0, pt_bt0, pt_W1, pt_b1, pt_g1, pt_bt1,
           out_W0, out_b0, out_g0, out_bt0, out_W1, out_b1, out_g1, out_bt1):
    src = set_indices[:, 0]
    dst = set_indices[:, 1]
    kidx = keypoint_indices[:, 0]
    kidx_pad = jnp.concatenate(
        [kidx, jnp.broadcast_to(kidx[-1:], (K_PAD - K_CENTERS,))])
    w0a = pt_W0[:D_FEAT]
    w0b_r = pt_W0[D_FEAT:].astype(jnp.bfloat16).astype(jnp.float32)

    t_tab = _prep(point_features, point_coordinates, w0a,
                  pt_b0.reshape(1, 64))
    c2 = _gatherc(t_tab, kidx_pad)
    x, st0 = _phase1(t_tab, c2, src, dst, w0b_r)

    s0 = jnp.sum(st0[:, 0, :], axis=0)
    m0 = s0[:64] / E
    v0 = s0[64:] / E - m0 * m0
    a0 = pt_g0 / jnp.sqrt(v0 + EPS)
    c0 = pt_bt0 - m0 * a0

    y4, st1 = _mlp1(x, pt_W1, pt_b1.reshape(1, 128),
                    a0.reshape(1, 64), c0.reshape(1, 64))

    m1 = st1[0] / E
    v1 = st1[1] / E - m1 * m1
    a1 = pt_g1 / jnp.sqrt(v1 + EPS)
    c1 = pt_bt1 - m1 * a1

    p_flat = _segmax(y4, dst)
    p = p_flat.reshape(SM_EG, SM_FG, K_CENTERS, FW)
    p = jnp.pad(p, ((0, 0), (0, 0), (0, K_PAD - K_CENTERS), (0, 0)))
    out = _mlp2(p, a1.reshape(1, 128), c1.reshape(1, 128),
                out_W0, out_b0.reshape(1, 128), out_g0.reshape(1, 128),
                out_bt0.reshape(1, 128),
                out_W1, out_b1.reshape(1, 128), out_g1.reshape(1, 128),
                out_bt1.reshape(1, 128))
    return out[:K_CENTERS]
